# baseline (speedup 1.0000x reference)
import functools

import jax
import jax.numpy as jnp
from jax import lax
from jax.experimental import pallas as pl
from jax.experimental.pallas import tpu as pltpu
from jax.experimental.pallas import tpu_sc as plsc

_N = 16384
_ROWS = 64
_COLS = 1_000_000
_L = 16
_BLK_C = 65536
_GRID = 16
_W = _GRID * _BLK_C
_NT = 16
_PER_T = _N // _NT
_WH = _W // 2
_PT_W = _WH // _NT
_ZCH = 8192


def _sc_counts(idx):
    mesh = plsc.VectorSubcoreMesh(core_axis_name="c", subcore_axis_name="s")

    @functools.partial(
        pl.kernel,
        mesh=mesh,
        out_type=jax.ShapeDtypeStruct((_W,), jnp.float32),
        scratch_types=[
            pltpu.VMEM((_PER_T,), jnp.int32),
            pltpu.VMEM((_ZCH,), jnp.float32),
            pltpu.VMEM((128,), jnp.float32),
            pltpu.VMEM((128,), jnp.float32),
            pltpu.VMEM_SHARED((_WH + 128,), jnp.float32),
        ],
    )
    def body(idx_hbm, w_hbm, idx_v, zero_v, one_v, flush_v, shared):
        core = lax.axis_index("c")
        tid = lax.axis_index("s")

        def fill(ref, n, val):
            def _f(i, carry):
                ref[pl.ds(i * _L, _L)] = jnp.full((_L,), val, jnp.float32)
                return carry
            lax.fori_loop(0, n // _L, _f, 0)

        fill(zero_v, _ZCH, 0.0)
        fill(one_v, 128, 1.0)
        base = tid * _PT_W

        def zchunk(k, carry):
            pltpu.sync_copy(zero_v, shared.at[pl.ds(base + k * _ZCH, _ZCH)])
            return carry
        lax.fori_loop(0, _PT_W // _ZCH, zchunk, 0)

        @pl.when(tid == 0)
        def _():
            pltpu.sync_copy(zero_v.at[pl.ds(0, 128)],
                            shared.at[pl.ds(_WH, 128)])

        pltpu.sync_copy(idx_hbm.at[pl.ds(tid * _PER_T, _PER_T)], idx_v)
        lo = core * _WH

        def localize(i, carry):
            sl = pl.ds(i * _L, _L)
            v = idx_v[sl] - lo
            ok = (v >= 0) & (v < _WH)
            idx_v[sl] = jnp.where(ok, v, _WH)
            return carry
        lax.fori_loop(0, _PER_T // _L, localize, 0)
        pltpu.sync_copy(shared.at[pl.ds(base, 128)], flush_v)
        plsc.subcore_barrier()

        def scatter(c, carry):
            pltpu.sync_copy(
                one_v, shared.at[idx_v.at[pl.ds(c * 128, 128)]], add=True)
            return carry
        lax.fori_loop(0, _PER_T // 128, scatter, 0)
        pltpu.sync_copy(shared.at[pl.ds(base, 128)], flush_v)
        plsc.subcore_barrier()

        def wchunk(k, carry):
            pltpu.sync_copy(
                shared.at[pl.ds(base + k * _ZCH, _ZCH)],
                w_hbm.at[pl.ds(lo + base + k * _ZCH, _ZCH)])
            return carry
        lax.fori_loop(0, _PT_W // _ZCH, wchunk, 0)

    return body(idx)


def _tc_matvec(scdata, w):
    def body(sc_ref, w_ref, out_ref):
        pid = pl.program_id(0)
        part = lax.dot_general(
            sc_ref[...], w_ref[...], (((1,), (0,)), ((), ())),
            preferred_element_type=jnp.float32)

        @pl.when(pid == 0)
        def _():
            out_ref[...] = jnp.zeros_like(out_ref)

        out_ref[...] += part

    return pl.pallas_call(
        body,
        grid=(_GRID,),
        in_specs=[
            pl.BlockSpec((_ROWS, _BLK_C), lambda i: (0, i)),
            pl.BlockSpec((_BLK_C,), lambda i: (i,)),
        ],
        out_specs=pl.BlockSpec((_ROWS,), lambda i: (0,)),
        out_shape=jax.ShapeDtypeStruct((_ROWS,), jnp.float32),
    )(scdata, w)


def kernel(scdata, inputs):
    idx = jax.random.randint(
        jax.random.key(1), (_N,), 0, scdata.shape[1] - 1, dtype=jnp.int32)
    idx = idx + (jnp.asarray(inputs, dtype=jnp.int32) - jnp.int32(_N))
    w = _sc_counts(idx)
    return _tc_matvec(scdata, w)


# speedup vs baseline: 1.0195x; 1.0195x over previous
import functools

import jax
import jax.numpy as jnp
from jax import lax
from jax.experimental import pallas as pl
from jax.experimental.pallas import tpu as pltpu
from jax.experimental.pallas import tpu_sc as plsc

_N = 16384
_ROWS = 64
_COLS = 1_000_000
_L = 16
_BLK_C = 32768
_GRID = 31
_W = _GRID * _BLK_C
_NT = 16
_PER_T = _N // _NT
_WH = _W // 2
_PT_W = _WH // _NT
_ZCH = 7936


def _sc_counts(idx):
    mesh = plsc.VectorSubcoreMesh(core_axis_name="c", subcore_axis_name="s")

    @functools.partial(
        pl.kernel,
        mesh=mesh,
        out_type=jax.ShapeDtypeStruct((_W,), jnp.float32),
        scratch_types=[
            pltpu.VMEM((_PER_T,), jnp.int32),
            pltpu.VMEM((_ZCH,), jnp.float32),
            pltpu.VMEM((128,), jnp.float32),
            pltpu.VMEM((128,), jnp.float32),
            pltpu.VMEM_SHARED((_WH + 128,), jnp.float32),
        ],
    )
    def body(idx_hbm, w_hbm, idx_v, zero_v, one_v, flush_v, shared):
        core = lax.axis_index("c")
        tid = lax.axis_index("s")

        def fill(ref, n, val):
            def _f(i, carry):
                ref[pl.ds(i * _L, _L)] = jnp.full((_L,), val, jnp.float32)
                return carry
            lax.fori_loop(0, n // _L, _f, 0)

        fill(zero_v, _ZCH, 0.0)
        fill(one_v, 128, 1.0)
        base = tid * _PT_W

        def zchunk(k, carry):
            pltpu.sync_copy(zero_v, shared.at[pl.ds(base + k * _ZCH, _ZCH)])
            return carry
        lax.fori_loop(0, _PT_W // _ZCH, zchunk, 0)

        @pl.when(tid == 0)
        def _():
            pltpu.sync_copy(zero_v.at[pl.ds(0, 128)],
                            shared.at[pl.ds(_WH, 128)])

        pltpu.sync_copy(idx_hbm.at[pl.ds(tid * _PER_T, _PER_T)], idx_v)
        lo = core * _WH

        def localize(i, carry):
            sl = pl.ds(i * _L, _L)
            v = idx_v[sl] - lo
            ok = (v >= 0) & (v < _WH)
            idx_v[sl] = jnp.where(ok, v, _WH)
            return carry
        lax.fori_loop(0, _PER_T // _L, localize, 0)
        pltpu.sync_copy(shared.at[pl.ds(base, 128)], flush_v)
        plsc.subcore_barrier()

        def scatter(c, carry):
            pltpu.sync_copy(
                one_v, shared.at[idx_v.at[pl.ds(c * 128, 128)]], add=True)
            return carry
        lax.fori_loop(0, _PER_T // 128, scatter, 0)
        pltpu.sync_copy(shared.at[pl.ds(base, 128)], flush_v)
        plsc.subcore_barrier()

        def wchunk(k, carry):
            pltpu.sync_copy(
                shared.at[pl.ds(base + k * _ZCH, _ZCH)],
                w_hbm.at[pl.ds(lo + base + k * _ZCH, _ZCH)])
            return carry
        lax.fori_loop(0, _PT_W // _ZCH, wchunk, 0)

    return body(idx)


def _tc_matvec(scdata, w):
    def body(sc_ref, w_ref, out_ref):
        pid = pl.program_id(0)
        part = lax.dot_general(
            sc_ref[...], w_ref[...], (((1,), (0,)), ((), ())),
            preferred_element_type=jnp.float32)

        @pl.when(pid == 0)
        def _():
            out_ref[...] = jnp.zeros_like(out_ref)

        out_ref[...] += part

    return pl.pallas_call(
        body,
        grid=(_GRID,),
        in_specs=[
            pl.BlockSpec((_ROWS, _BLK_C), lambda i: (0, i)),
            pl.BlockSpec((_BLK_C,), lambda i: (i,)),
        ],
        out_specs=pl.BlockSpec((_ROWS,), lambda i: (0,)),
        out_shape=jax.ShapeDtypeStruct((_ROWS,), jnp.float32),
    )(scdata, w)


def kernel(scdata, inputs):
    idx = jax.random.randint(
        jax.random.key(1), (_N,), 0, scdata.shape[1] - 1, dtype=jnp.int32)
    idx = idx + (jnp.asarray(inputs, dtype=jnp.int32) - jnp.int32(_N))
    w = _sc_counts(idx)
    return _tc_matvec(scdata, w)


# speedup vs baseline: 1.0766x; 1.0561x over previous
import functools

import jax
import jax.numpy as jnp
from jax import lax
from jax.experimental import pallas as pl
from jax.experimental.pallas import tpu as pltpu
from jax.experimental.pallas import tpu_sc as plsc

_N = 16384
_ROWS = 64
_COLS = 1_000_000
_L = 16
_BLK_C = 32768
_GRID = 31
_W = _GRID * _BLK_C
_NT = 16
_PER_T = _N // _NT
_WH = _W // 2
_PT_W = _WH // _NT
_ZCH = 7936


def _sc_counts(idx):
    mesh = plsc.VectorSubcoreMesh(core_axis_name="c", subcore_axis_name="s")

    @functools.partial(
        pl.kernel,
        mesh=mesh,
        out_type=jax.ShapeDtypeStruct((_W,), jnp.float32),
        scratch_types=[
            pltpu.VMEM((_PER_T,), jnp.int32),
            pltpu.VMEM((_ZCH,), jnp.float32),
            pltpu.VMEM((128,), jnp.float32),
            pltpu.VMEM((128,), jnp.float32),
            pltpu.VMEM_SHARED((_WH + 256,), jnp.float32),
        ],
    )
    def body(idx_hbm, w_hbm, idx_v, zero_v, one_v, flush_v, shared):
        core = lax.axis_index("c")
        tid = lax.axis_index("s")

        def fill(ref, n, val):
            def _f(i, carry):
                ref[pl.ds(i * _L, _L)] = jnp.full((_L,), val, jnp.float32)
                return carry
            lax.fori_loop(0, n // _L, _f, 0)

        fill(zero_v, _ZCH, 0.0)
        fill(one_v, 128, 1.0)
        base = tid * _PT_W

        def zchunk(k, carry):
            pltpu.sync_copy(zero_v, shared.at[pl.ds(base + k * _ZCH, _ZCH)])
            return carry
        lax.fori_loop(0, _PT_W // _ZCH, zchunk, 0)

        @pl.when(tid == 0)
        def _():
            pltpu.sync_copy(zero_v.at[pl.ds(0, 256)],
                            shared.at[pl.ds(_WH, 256)])

        pltpu.sync_copy(idx_hbm.at[pl.ds(tid * _PER_T, _PER_T)], idx_v)
        lo = core * _WH
        dump = _WH + tid * _L + lax.iota(jnp.int32, _L)

        def localize(i, carry):
            sl = pl.ds(i * _L, _L)
            v = idx_v[sl] - lo
            ok = (v >= 0) & (v < _WH)
            idx_v[sl] = jnp.where(ok, v, dump)
            return carry
        lax.fori_loop(0, _PER_T // _L, localize, 0)
        pltpu.sync_copy(shared.at[pl.ds(base, 128)], flush_v)
        plsc.subcore_barrier()

        def scatter(c, carry):
            pltpu.sync_copy(
                one_v, shared.at[idx_v.at[pl.ds(c * 128, 128)]], add=True)
            return carry
        lax.fori_loop(0, _PER_T // 128, scatter, 0)
        pltpu.sync_copy(shared.at[pl.ds(base, 128)], flush_v)
        plsc.subcore_barrier()

        def wchunk(k, carry):
            pltpu.sync_copy(
                shared.at[pl.ds(base + k * _ZCH, _ZCH)],
                w_hbm.at[pl.ds(lo + base + k * _ZCH, _ZCH)])
            return carry
        lax.fori_loop(0, _PT_W // _ZCH, wchunk, 0)

    return body(idx)


def _tc_matvec(scdata, w):
    def body(sc_ref, w_ref, out_ref):
        pid = pl.program_id(0)
        part = lax.dot_general(
            sc_ref[...], w_ref[...], (((1,), (0,)), ((), ())),
            preferred_element_type=jnp.float32)

        @pl.when(pid == 0)
        def _():
            out_ref[...] = jnp.zeros_like(out_ref)

        out_ref[...] += part

    return pl.pallas_call(
        body,
        grid=(_GRID,),
        in_specs=[
            pl.BlockSpec((_ROWS, _BLK_C), lambda i: (0, i)),
            pl.BlockSpec((_BLK_C,), lambda i: (i,)),
        ],
        out_specs=pl.BlockSpec((_ROWS,), lambda i: (0,)),
        out_shape=jax.ShapeDtypeStruct((_ROWS,), jnp.float32),
    )(scdata, w)


def kernel(scdata, inputs):
    idx = jax.random.randint(
        jax.random.key(1), (_N,), 0, scdata.shape[1] - 1, dtype=jnp.int32)
    idx = idx + (jnp.asarray(inputs, dtype=jnp.int32) - jnp.int32(_N))
    w = _sc_counts(idx)
    return _tc_matvec(scdata, w)
